# Initial kernel scaffold; baseline (speedup 1.0000x reference)
import functools

import jax
import jax.numpy as jnp
from jax import lax
from jax.experimental import pallas as pl
from jax.experimental.pallas import tpu as pltpu
from jax.experimental.pallas import tpu_sc as plsc

_N = 100000
_NPAD = 100352
_ROWS = _NPAD // 1024
_E = 6400000
_NW = 32
_EPW = _E // _NW
_CHUNK = 4000
_NCH = _EPW // _CHUNK
_GRP = _CHUNK // 16


def _act_body(v_ref, vh_ref, sl_ref, o_ref):
    o_ref[...] = jax.nn.sigmoid((v_ref[...] - vh_ref[...]) / sl_ref[...])


_sc_mesh = plsc.VectorSubcoreMesh(
    core_axis_name="c", subcore_axis_name="s", num_cores=2, num_subcores=16
)


@functools.partial(
    pl.kernel,
    out_type=(
        jax.ShapeDtypeStruct((_NW, _NPAD), jnp.float32),
        jax.ShapeDtypeStruct((_E,), jnp.float32),
    ),
    mesh=_sc_mesh,
    scratch_types=[
        pltpu.VMEM((_NPAD,), jnp.float32),
        pltpu.VMEM((_CHUNK,), jnp.int32),
        pltpu.VMEM((_CHUNK,), jnp.float32),
        pltpu.VMEM((_CHUNK,), jnp.float32),
    ],
)
def _sc_edges(act_hbm, src_hbm, dst_hbm, w_hbm, part_hbm, msg_hbm,
              table, idxbuf, wbuf, msgbuf):
    wid = lax.axis_index("c") * 16 + lax.axis_index("s")
    base = wid * _EPW

    pltpu.sync_copy(act_hbm, table)

    def p1_chunk(ci, _):
        off = base + ci * _CHUNK
        pltpu.sync_copy(src_hbm.at[pl.ds(off, _CHUNK)], idxbuf)
        pltpu.sync_copy(w_hbm.at[pl.ds(off, _CHUNK)], wbuf)

        def grp(gi, _):
            s = gi * 16
            idx = idxbuf[pl.ds(s, 16)]
            a = plsc.load_gather(table, [idx])
            msgbuf[pl.ds(s, 16)] = wbuf[pl.ds(s, 16)] * a
            return 0

        lax.fori_loop(0, _GRP, grp, 0)
        pltpu.sync_copy(msgbuf, msg_hbm.at[pl.ds(off, _CHUNK)])
        return 0

    lax.fori_loop(0, _NCH, p1_chunk, 0)

    zeros = jnp.zeros((16,), jnp.float32)

    def zero_grp(i, _):
        table[pl.ds(i * 16, 16)] = zeros
        return 0

    lax.fori_loop(0, _NPAD // 16, zero_grp, 0)

    def p2_chunk(ci, _):
        off = base + ci * _CHUNK
        pltpu.sync_copy(dst_hbm.at[pl.ds(off, _CHUNK)], idxbuf)
        pltpu.sync_copy(msg_hbm.at[pl.ds(off, _CHUNK)], msgbuf)

        def grp(gi, _):
            s = gi * 16
            d = idxbuf[pl.ds(s, 16)]
            m = msgbuf[pl.ds(s, 16)]
            plsc.addupdate_scatter(table, [d], m)
            return 0

        lax.fori_loop(0, _GRP, grp, 0)
        return 0

    lax.fori_loop(0, _NCH, p2_chunk, 0)
    pltpu.sync_copy(table, part_hbm.at[wid])


def _post_body(p_ref, v_ref, st_ref, m_ref, h_ref, n_ref, gna_ref, gk_ref,
               gl_ref, ena_ref, ek_ref, el_ref, ib_ref, ss_ref, c_ref, o_ref):
    I_syn = jnp.sum(p_ref[...], axis=0)
    v = v_ref[...]
    I_Na = gna_ref[...] * m_ref[...] ** 3 * h_ref[...] * (v - ena_ref[...])
    I_K = gk_ref[...] * n_ref[...] ** 4 * (v - ek_ref[...])
    I_L = gl_ref[...] * (v - el_ref[...])
    I_ext = ib_ref[...] + ss_ref[...] * st_ref[...]
    o_ref[...] = (-I_Na - I_K - I_L + I_syn + I_ext) / c_ref[...]


def kernel(voltage, stimulus, hh_m, hh_h, hh_n, edge_index, W, syn_v_half,
           syn_slope, g_Na, g_K, g_L, E_Na, E_K, E_L, I_bias, stim_scale, C):
    f32 = jnp.float32
    src = edge_index[0].astype(jnp.int32)
    dst = edge_index[1].astype(jnp.int32)
    pad = _NPAD - _N

    def pad2(x, val=0.0):
        return jnp.pad(x.astype(f32), (0, pad), constant_values=val).reshape(
            _ROWS, 1024)

    v2 = pad2(voltage)
    vh2 = pad2(syn_v_half)
    sl2 = pad2(syn_slope, 1.0)

    act = pl.pallas_call(
        _act_body,
        out_shape=jax.ShapeDtypeStruct((_ROWS, 1024), f32),
    )(v2, vh2, sl2)

    partials, _ = _sc_edges(act.reshape(_NPAD), src, dst, W)

    dv2 = pl.pallas_call(
        _post_body,
        out_shape=jax.ShapeDtypeStruct((_ROWS, 1024), f32),
    )(partials.reshape(_NW, _ROWS, 1024), v2, pad2(stimulus), pad2(hh_m),
      pad2(hh_h), pad2(hh_n), pad2(g_Na), pad2(g_K), pad2(g_L), pad2(E_Na),
      pad2(E_K), pad2(E_L), pad2(I_bias), pad2(stim_scale), pad2(C, 1.0))

    return dv2.reshape(_NPAD)[:_N, None]


# speedup vs baseline: 401.1112x; 401.1112x over previous
import functools

import jax
import jax.numpy as jnp
from jax import lax
from jax.experimental import pallas as pl
from jax.experimental.pallas import tpu as pltpu
from jax.experimental.pallas import tpu_sc as plsc

_N = 100000
_NPAD = 100352
_ROWS = _NPAD // 1024
_E = 6400000
_NW = 32
_EPW = _E // _NW
_CHUNK = 4000
_NCH = _EPW // _CHUNK
_GRP = _CHUNK // 16


def _act_body(v_ref, vh_ref, sl_ref, o_ref):
    o_ref[...] = jax.nn.sigmoid((v_ref[...] - vh_ref[...]) / sl_ref[...])


_sc_mesh = plsc.VectorSubcoreMesh(
    core_axis_name="c", subcore_axis_name="s", num_cores=2, num_subcores=16
)


@functools.partial(
    pl.kernel,
    out_type=(
        jax.ShapeDtypeStruct((_NW, _NPAD), jnp.float32),
        jax.ShapeDtypeStruct((_E,), jnp.float32),
    ),
    mesh=_sc_mesh,
    scratch_types=[
        pltpu.VMEM((_NPAD,), jnp.float32),
        pltpu.VMEM((_CHUNK,), jnp.int32),
        pltpu.VMEM((_CHUNK,), jnp.float32),
        pltpu.VMEM((_CHUNK,), jnp.float32),
    ],
    compiler_params=pltpu.CompilerParams(needs_layout_passes=False),
)
def _sc_edges(act_hbm, src_hbm, dst_hbm, w_hbm, part_hbm, msg_hbm,
              table, idxbuf, wbuf, msgbuf):
    wid = lax.axis_index("c") * 16 + lax.axis_index("s")
    base = wid * _EPW

    pltpu.sync_copy(act_hbm, table)

    def p1_chunk(ci, _):
        off = base + ci * _CHUNK
        pltpu.sync_copy(src_hbm.at[pl.ds(off, _CHUNK)], idxbuf)
        pltpu.sync_copy(w_hbm.at[pl.ds(off, _CHUNK)], wbuf)

        def grp(gi, _):
            s = gi * 16
            idx = idxbuf[pl.ds(s, 16)]
            a = plsc.load_gather(table, [idx])
            msgbuf[pl.ds(s, 16)] = wbuf[pl.ds(s, 16)] * a
            return 0

        lax.fori_loop(0, _GRP, grp, 0)
        pltpu.sync_copy(msgbuf, msg_hbm.at[pl.ds(off, _CHUNK)])
        return 0

    lax.fori_loop(0, _NCH, p1_chunk, 0)

    zeros = jnp.zeros((16,), jnp.float32)

    def zero_grp(i, _):
        table[pl.ds(i * 16, 16)] = zeros
        return 0

    lax.fori_loop(0, _NPAD // 16, zero_grp, 0)

    def p2_chunk(ci, _):
        off = base + ci * _CHUNK
        pltpu.sync_copy(dst_hbm.at[pl.ds(off, _CHUNK)], idxbuf)
        pltpu.sync_copy(msg_hbm.at[pl.ds(off, _CHUNK)], msgbuf)

        def grp(gi, _):
            s = gi * 16
            d = idxbuf[pl.ds(s, 16)]
            m = msgbuf[pl.ds(s, 16)]
            plsc.addupdate_scatter(table, [d], m)
            return 0

        lax.fori_loop(0, _GRP, grp, 0)
        return 0

    lax.fori_loop(0, _NCH, p2_chunk, 0)
    pltpu.sync_copy(table, part_hbm.at[wid])


def _post_body(p_ref, v_ref, st_ref, m_ref, h_ref, n_ref, gna_ref, gk_ref,
               gl_ref, ena_ref, ek_ref, el_ref, ib_ref, ss_ref, c_ref, o_ref):
    I_syn = jnp.sum(p_ref[...], axis=0)
    v = v_ref[...]
    I_Na = gna_ref[...] * m_ref[...] ** 3 * h_ref[...] * (v - ena_ref[...])
    I_K = gk_ref[...] * n_ref[...] ** 4 * (v - ek_ref[...])
    I_L = gl_ref[...] * (v - el_ref[...])
    I_ext = ib_ref[...] + ss_ref[...] * st_ref[...]
    o_ref[...] = (-I_Na - I_K - I_L + I_syn + I_ext) / c_ref[...]


def kernel(voltage, stimulus, hh_m, hh_h, hh_n, edge_index, W, syn_v_half,
           syn_slope, g_Na, g_K, g_L, E_Na, E_K, E_L, I_bias, stim_scale, C):
    f32 = jnp.float32
    src = edge_index[0].astype(jnp.int32)
    dst = edge_index[1].astype(jnp.int32)
    pad = _NPAD - _N

    def pad2(x, val=0.0):
        return jnp.pad(x.astype(f32), (0, pad), constant_values=val).reshape(
            _ROWS, 1024)

    v2 = pad2(voltage)
    vh2 = pad2(syn_v_half)
    sl2 = pad2(syn_slope, 1.0)

    act = pl.pallas_call(
        _act_body,
        out_shape=jax.ShapeDtypeStruct((_ROWS, 1024), f32),
    )(v2, vh2, sl2)

    partials, _ = _sc_edges(act.reshape(_NPAD), src, dst, W)

    dv2 = pl.pallas_call(
        _post_body,
        out_shape=jax.ShapeDtypeStruct((_ROWS, 1024), f32),
    )(partials.reshape(_NW, _ROWS, 1024), v2, pad2(stimulus), pad2(hh_m),
      pad2(hh_h), pad2(hh_n), pad2(g_Na), pad2(g_K), pad2(g_L), pad2(E_Na),
      pad2(E_K), pad2(E_L), pad2(I_bias), pad2(stim_scale), pad2(C, 1.0))

    return dv2.reshape(_NPAD)[:_N, None]
